# baseline (speedup 1.0000x reference)
import jax
import jax.numpy as jnp
from jax.experimental import pallas as pl

N = 4096
T = 256
NT = N // T
JT = 1024
TB = 512
C = 256
KF = 512
NEG = -1e30


def _dist_kernel(f_ref, d_ref, mx_ref):
    i = pl.program_id(0)
    j = pl.program_id(1)
    a = f_ref[pl.ds(i * T, T), :]
    b = f_ref[pl.ds(j * JT, JT), :]
    xi2 = jnp.sum(a * a, axis=1)
    xj2 = jnp.sum(b * b, axis=1)
    g = jax.lax.dot_general(a, b, (((1,), (1,)), ((), ())),
                            preferred_element_type=jnp.float32)
    dist = xi2[:, None] + xj2[None, :] - 2.0 * g
    d_ref[...] = dist
    m = jnp.max(dist)
    first = (i == 0) & (j == 0)

    @pl.when(first)
    def _():
        mx_ref[...] = jnp.full((8, 128), m, jnp.float32)

    @pl.when(jnp.logical_not(first))
    def _():
        mx_ref[...] = jnp.maximum(mx_ref[...], m)


def _mask_kernel(t_ref, d_ref, mt_ref, dv_ref):
    b = pl.program_id(0)
    t = t_ref[0, 0]
    d = d_ref[...]
    gb = b * T + jax.lax.broadcasted_iota(jnp.int32, (T, N), 0)
    ga = jax.lax.broadcasted_iota(jnp.int32, (T, N), 1)
    mf = ((d < t) & (ga < gb)).astype(jnp.float32)
    mt_ref[...] = mf.astype(jnp.bfloat16)
    dv = jax.lax.rsqrt(1.0 + jnp.sum(mf, axis=1))
    dv_ref[...] = jnp.broadcast_to(dv[None, :], (8, T))


def _lin_kernel(s_ref, dv_ref, x_ref, w_ref, o_ref, u_ref):
    i = pl.program_id(0)
    s = s_ref[0, pl.ds(i * T, T)][:, None]
    dv = dv_ref[0, pl.ds(i * T, T)][:, None]
    xw = jnp.dot(x_ref[...] * s, w_ref[...], preferred_element_type=jnp.float32)
    o_ref[...] = xw
    u_ref[...] = (xw * dv).astype(jnp.bfloat16)


def _aggconv_kernel(mt_ref, u_ref, xwb_ref, dv_ref, wp_ref, b_ref,
                    o_ref, sw_ref):
    b = pl.program_id(0)
    acc = jnp.dot(mt_ref[...], u_ref[...], preferred_element_type=jnp.float32)
    dvb = dv_ref[0, pl.ds(b * TB, TB)][:, None]
    z = dvb * acc + dvb * dvb * xwb_ref[...] + b_ref[0, :][None, :]
    h = jnp.maximum(z, 0.0)
    o_ref[...] = h
    sw = jnp.sum(h * wp_ref[0, :][None, :], axis=1)
    sw_ref[...] = jnp.broadcast_to(sw[None, :], (8, TB))


def _aggscore_kernel(mt_ref, sw_ref, dv_ref, bp_ref, o_ref):
    b = pl.program_id(0)
    u = sw_ref[0, :] * dv_ref[0, :]
    acc = jnp.sum(mt_ref[...].astype(jnp.float32) * u[None, :], axis=1)
    dvb = dv_ref[0, pl.ds(b * TB, TB)]
    swb = sw_ref[0, pl.ds(b * TB, TB)]
    z = dvb * acc + dvb * dvb * swb + bp_ref[0, 0]
    z = jnp.where(dvb > 0.0, z, NEG)
    o_ref[...] = jnp.broadcast_to(z[None, :], (8, TB))


def _aggdeg_kernel(mt_ref, kp_ref, o_ref):
    b = pl.program_id(0)
    u = kp_ref[0, :]
    acc = jnp.sum(mt_ref[...].astype(jnp.float32) * u[None, :], axis=1)
    dv = kp_ref[0, pl.ds(b * TB, TB)] * jax.lax.rsqrt(1.0 + acc)
    o_ref[...] = jnp.broadcast_to(dv[None, :], (8, TB))


def _readout_kernel(h_ref, s_ref, k_ref, mx_ref, sm_ref):
    i = pl.program_id(0)
    s = s_ref[0, pl.ds(i * T, T)][:, None]
    keep = k_ref[0, pl.ds(i * T, T)][:, None] > 0.0
    x = h_ref[...] * s
    tmax = jnp.max(jnp.where(keep, x, NEG), axis=0)
    tsum = jnp.sum(jnp.where(keep, x, 0.0), axis=0)

    @pl.when(i == 0)
    def _():
        mx_ref[...] = jnp.full_like(mx_ref, NEG)
        sm_ref[...] = jnp.zeros_like(sm_ref)

    mx_ref[...] = jnp.maximum(mx_ref[...], tmax[None, :])
    sm_ref[...] += tsum[None, :]


def _dist(fp):
    return pl.pallas_call(
        _dist_kernel,
        grid=(NT, N // JT),
        in_specs=[pl.BlockSpec((N, KF), lambda i, j: (0, 0))],
        out_specs=[pl.BlockSpec((T, JT), lambda i, j: (i, j)),
                   pl.BlockSpec((8, 128), lambda i, j: (0, 0))],
        out_shape=[jax.ShapeDtypeStruct((N, N), jnp.float32),
                   jax.ShapeDtypeStruct((8, 128), jnp.float32)],
    )(fp)


def _mask(d, t):
    tb = jnp.broadcast_to(jnp.reshape(t, (1, 1)), (8, 128))
    return pl.pallas_call(
        _mask_kernel,
        grid=(NT,),
        in_specs=[pl.BlockSpec((8, 128), lambda b: (0, 0)),
                  pl.BlockSpec((T, N), lambda b: (b, 0))],
        out_specs=[pl.BlockSpec((T, N), lambda b: (b, 0)),
                   pl.BlockSpec((8, T), lambda b: (0, b))],
        out_shape=[jax.ShapeDtypeStruct((N, N), jnp.bfloat16),
                   jax.ShapeDtypeStruct((8, N), jnp.float32)],
    )(tb, d)


def _lin(x, w, s, dv):
    kd, m = w.shape
    return pl.pallas_call(
        _lin_kernel,
        grid=(NT,),
        in_specs=[pl.BlockSpec((8, N), lambda i: (0, 0)),
                  pl.BlockSpec((8, N), lambda i: (0, 0)),
                  pl.BlockSpec((T, kd), lambda i: (i, 0)),
                  pl.BlockSpec((kd, m), lambda i: (0, 0))],
        out_specs=[pl.BlockSpec((T, m), lambda i: (i, 0)),
                   pl.BlockSpec((T, m), lambda i: (i, 0))],
        out_shape=[jax.ShapeDtypeStruct((N, m), jnp.float32),
                   jax.ShapeDtypeStruct((N, m), jnp.bfloat16)],
    )(s, dv, x, w)


def _aggconv(mt, u, xw, dv, wp, bias):
    wpr = jnp.broadcast_to(wp[:, 0][None, :], (8, C))
    bb = jnp.broadcast_to(bias[None, :], (8, C))
    return pl.pallas_call(
        _aggconv_kernel,
        grid=(N // TB,),
        in_specs=[pl.BlockSpec((TB, N), lambda b: (b, 0)),
                  pl.BlockSpec((N, C), lambda b: (0, 0)),
                  pl.BlockSpec((TB, C), lambda b: (b, 0)),
                  pl.BlockSpec((8, N), lambda b: (0, 0)),
                  pl.BlockSpec((8, C), lambda b: (0, 0)),
                  pl.BlockSpec((8, C), lambda b: (0, 0))],
        out_specs=[pl.BlockSpec((TB, C), lambda b: (b, 0)),
                   pl.BlockSpec((8, TB), lambda b: (0, b))],
        out_shape=[jax.ShapeDtypeStruct((N, C), jnp.float32),
                   jax.ShapeDtypeStruct((8, N), jnp.float32)],
    )(mt, u, xw, dv, wpr, bb)


def _aggscore(mt, sw, dv, bp):
    bpb = jnp.broadcast_to(jnp.reshape(bp, (1, 1)), (8, 128))
    return pl.pallas_call(
        _aggscore_kernel,
        grid=(N // TB,),
        in_specs=[pl.BlockSpec((TB, N), lambda b: (b, 0)),
                  pl.BlockSpec((8, N), lambda b: (0, 0)),
                  pl.BlockSpec((8, N), lambda b: (0, 0)),
                  pl.BlockSpec((8, 128), lambda b: (0, 0))],
        out_specs=pl.BlockSpec((8, TB), lambda b: (0, b)),
        out_shape=jax.ShapeDtypeStruct((8, N), jnp.float32),
    )(mt, sw, dv, bpb)


def _aggdeg(mt, kp):
    return pl.pallas_call(
        _aggdeg_kernel,
        grid=(N // TB,),
        in_specs=[pl.BlockSpec((TB, N), lambda b: (b, 0)),
                  pl.BlockSpec((8, N), lambda b: (0, 0))],
        out_specs=pl.BlockSpec((8, TB), lambda b: (0, b)),
        out_shape=jax.ShapeDtypeStruct((8, N), jnp.float32),
    )(mt, kp)


def _readout(h, s, kp, k):
    mx, sm = pl.pallas_call(
        _readout_kernel,
        grid=(NT,),
        in_specs=[pl.BlockSpec((T, C), lambda i: (i, 0)),
                  pl.BlockSpec((8, N), lambda i: (0, 0)),
                  pl.BlockSpec((8, N), lambda i: (0, 0))],
        out_specs=[pl.BlockSpec((8, C), lambda i: (0, 0)),
                   pl.BlockSpec((8, C), lambda i: (0, 0))],
        out_shape=[jax.ShapeDtypeStruct((8, C), jnp.float32),
                   jax.ShapeDtypeStruct((8, C), jnp.float32)],
    )(h, s, kp)
    return jnp.concatenate([mx[0], sm[0] / k])


def _layer(mt, x_in, w, b, wp, bp, s_in, dv, kx):
    xw, u = _lin(x_in, w, s_in, dv)
    h, sw = _aggconv(mt, u, xw, dv, wp, b)
    sc = _aggscore(mt, sw, dv, bp)[0]
    tau = jax.lax.top_k(sc, kx)[0][kx - 1]
    keep_n = (sc >= tau).astype(jnp.float32)
    s_n = keep_n * jnp.tanh(sc)
    s8 = jnp.broadcast_to(s_n[None, :], (8, N))
    k8 = jnp.broadcast_to(keep_n[None, :], (8, N))
    xr = _readout(h, s8, k8, kx)
    return h, s8, k8, xr


def kernel(feature, W1, b1, Wp1, bp1, W2, b2, Wp2, bp2, W3, b3, Wp3, bp3):
    f32 = jnp.float32
    fp = jnp.pad(feature.astype(f32), ((0, 0), (0, KF - feature.shape[1])))
    w1p = jnp.pad(W1, ((0, KF - W1.shape[0]), (0, 0)))

    d, mx = _dist(fp)
    t = 0.5 * mx[0, 0]
    mt, dv1 = _mask(d, t)

    ones8 = jnp.ones((8, N), f32)
    k1, k2, k3 = 3072, 2304, 1728

    h1, s1, kp2, x1r = _layer(mt, fp, w1p, b1, Wp1, bp1, ones8, dv1, k1)
    dv2 = _aggdeg(mt, kp2)
    h2, s2, kp3, x2r = _layer(mt, h1, W2, b2, Wp2, bp2, s1, dv2, k2)
    dv3 = _aggdeg(mt, kp3)
    _, _, _, x3r = _layer(mt, h2, W3, b3, Wp3, bp3, s2, dv3, k3)

    return (x1r + x2r + x3r)[None, :]


# speedup vs baseline: 1.0507x; 1.0507x over previous
import jax
import jax.numpy as jnp
from jax.experimental import pallas as pl

N = 4096
T = 256
NT = N // T
JT = 1024
TB = 512
C = 256
KF = 512
NEG = -1e30


def _dist_kernel(f_ref, d_ref, mx_ref):
    i = pl.program_id(0)
    j = pl.program_id(1)
    a = f_ref[pl.ds(i * T, T), :]
    b = f_ref[pl.ds(j * JT, JT), :]
    xi2 = jnp.sum(a * a, axis=1)
    xj2 = jnp.sum(b * b, axis=1)
    g = jax.lax.dot_general(a, b, (((1,), (1,)), ((), ())),
                            preferred_element_type=jnp.float32)
    dist = xi2[:, None] + xj2[None, :] - 2.0 * g
    d_ref[...] = dist
    m = jnp.max(dist)
    first = (i == 0) & (j == 0)

    @pl.when(first)
    def _():
        mx_ref[...] = jnp.full((8, 128), m, jnp.float32)

    @pl.when(jnp.logical_not(first))
    def _():
        mx_ref[...] = jnp.maximum(mx_ref[...], m)


def _mask_kernel(t_ref, d_ref, mt_ref, dv_ref):
    b = pl.program_id(0)
    t = t_ref[0, 0]
    d = d_ref[...]
    gb = b * T + jax.lax.broadcasted_iota(jnp.int32, (T, N), 0)
    ga = jax.lax.broadcasted_iota(jnp.int32, (T, N), 1)
    mf = ((d < t) & (ga < gb)).astype(jnp.float32)
    mt_ref[...] = mf.astype(jnp.bfloat16)
    dv = jax.lax.rsqrt(1.0 + jnp.sum(mf, axis=1))
    dv_ref[...] = jnp.broadcast_to(dv[None, :], (8, T))


def _lin_kernel(s_ref, dv_ref, kp_ref, x_ref, w_ref, o_ref, u_ref,
                mx_ref, sm_ref):
    i = pl.program_id(0)
    s = s_ref[0, pl.ds(i * T, T)][:, None]
    dv = dv_ref[0, pl.ds(i * T, T)][:, None]
    xs = x_ref[...] * s
    xw = jnp.dot(xs, w_ref[...], preferred_element_type=jnp.float32)
    o_ref[...] = xw
    u_ref[...] = (xw * dv).astype(jnp.bfloat16)
    keep = kp_ref[0, pl.ds(i * T, T)][:, None] > 0.0
    tmax = jnp.max(jnp.where(keep, xs, NEG), axis=0)
    tsum = jnp.sum(jnp.where(keep, xs, 0.0), axis=0)

    @pl.when(i == 0)
    def _():
        mx_ref[...] = jnp.full_like(mx_ref, NEG)
        sm_ref[...] = jnp.zeros_like(sm_ref)

    mx_ref[...] = jnp.maximum(mx_ref[...], tmax[None, :])
    sm_ref[...] += tsum[None, :]


def _aggconv_kernel(mt_ref, u_ref, xwb_ref, dv_ref, wp_ref, b_ref,
                    o_ref, sw_ref):
    b = pl.program_id(0)
    acc = jnp.dot(mt_ref[...], u_ref[...], preferred_element_type=jnp.float32)
    dvb = dv_ref[0, pl.ds(b * TB, TB)][:, None]
    z = dvb * acc + dvb * dvb * xwb_ref[...] + b_ref[0, :][None, :]
    h = jnp.maximum(z, 0.0)
    o_ref[...] = h
    sw = jnp.sum(h * wp_ref[0, :][None, :], axis=1)
    sw_ref[...] = jnp.broadcast_to(sw[None, :], (8, TB))


def _aggscore_kernel(mt_ref, sw_ref, dv_ref, bp_ref, o_ref):
    b = pl.program_id(0)
    u = sw_ref[0, :] * dv_ref[0, :]
    acc = jnp.sum(mt_ref[...].astype(jnp.float32) * u[None, :], axis=1)
    dvb = dv_ref[0, pl.ds(b * TB, TB)]
    swb = sw_ref[0, pl.ds(b * TB, TB)]
    z = dvb * acc + dvb * dvb * swb + bp_ref[0, 0]
    z = jnp.where(dvb > 0.0, z, NEG)
    o_ref[...] = jnp.broadcast_to(z[None, :], (8, TB))


def _aggdeg_kernel(mt_ref, kp_ref, o_ref):
    b = pl.program_id(0)
    u = kp_ref[0, :]
    acc = jnp.sum(mt_ref[...].astype(jnp.float32) * u[None, :], axis=1)
    dv = kp_ref[0, pl.ds(b * TB, TB)] * jax.lax.rsqrt(1.0 + acc)
    o_ref[...] = jnp.broadcast_to(dv[None, :], (8, TB))


def _readout_kernel(h_ref, s_ref, k_ref, mx_ref, sm_ref):
    i = pl.program_id(0)
    s = s_ref[0, pl.ds(i * T, T)][:, None]
    keep = k_ref[0, pl.ds(i * T, T)][:, None] > 0.0
    x = h_ref[...] * s
    tmax = jnp.max(jnp.where(keep, x, NEG), axis=0)
    tsum = jnp.sum(jnp.where(keep, x, 0.0), axis=0)

    @pl.when(i == 0)
    def _():
        mx_ref[...] = jnp.full_like(mx_ref, NEG)
        sm_ref[...] = jnp.zeros_like(sm_ref)

    mx_ref[...] = jnp.maximum(mx_ref[...], tmax[None, :])
    sm_ref[...] += tsum[None, :]


def _dist(fp):
    return pl.pallas_call(
        _dist_kernel,
        grid=(NT, N // JT),
        in_specs=[pl.BlockSpec((N, KF), lambda i, j: (0, 0))],
        out_specs=[pl.BlockSpec((T, JT), lambda i, j: (i, j)),
                   pl.BlockSpec((8, 128), lambda i, j: (0, 0))],
        out_shape=[jax.ShapeDtypeStruct((N, N), jnp.float32),
                   jax.ShapeDtypeStruct((8, 128), jnp.float32)],
    )(fp)


def _mask(d, t):
    tb = jnp.broadcast_to(jnp.reshape(t, (1, 1)), (8, 128))
    return pl.pallas_call(
        _mask_kernel,
        grid=(NT,),
        in_specs=[pl.BlockSpec((8, 128), lambda b: (0, 0)),
                  pl.BlockSpec((T, N), lambda b: (b, 0))],
        out_specs=[pl.BlockSpec((T, N), lambda b: (b, 0)),
                   pl.BlockSpec((8, T), lambda b: (0, b))],
        out_shape=[jax.ShapeDtypeStruct((N, N), jnp.bfloat16),
                   jax.ShapeDtypeStruct((8, N), jnp.float32)],
    )(tb, d)


def _lin(x, w, s, dv, kp):
    kd, m = w.shape
    return pl.pallas_call(
        _lin_kernel,
        grid=(NT,),
        in_specs=[pl.BlockSpec((8, N), lambda i: (0, 0)),
                  pl.BlockSpec((8, N), lambda i: (0, 0)),
                  pl.BlockSpec((8, N), lambda i: (0, 0)),
                  pl.BlockSpec((T, kd), lambda i: (i, 0)),
                  pl.BlockSpec((kd, m), lambda i: (0, 0))],
        out_specs=[pl.BlockSpec((T, m), lambda i: (i, 0)),
                   pl.BlockSpec((T, m), lambda i: (i, 0)),
                   pl.BlockSpec((8, kd), lambda i: (0, 0)),
                   pl.BlockSpec((8, kd), lambda i: (0, 0))],
        out_shape=[jax.ShapeDtypeStruct((N, m), jnp.float32),
                   jax.ShapeDtypeStruct((N, m), jnp.bfloat16),
                   jax.ShapeDtypeStruct((8, kd), jnp.float32),
                   jax.ShapeDtypeStruct((8, kd), jnp.float32)],
    )(s, dv, kp, x, w)


def _aggconv(mt, u, xw, dv, wp, bias):
    wpr = jnp.broadcast_to(wp[:, 0][None, :], (8, C))
    bb = jnp.broadcast_to(bias[None, :], (8, C))
    return pl.pallas_call(
        _aggconv_kernel,
        grid=(N // TB,),
        in_specs=[pl.BlockSpec((TB, N), lambda b: (b, 0)),
                  pl.BlockSpec((N, C), lambda b: (0, 0)),
                  pl.BlockSpec((TB, C), lambda b: (b, 0)),
                  pl.BlockSpec((8, N), lambda b: (0, 0)),
                  pl.BlockSpec((8, C), lambda b: (0, 0)),
                  pl.BlockSpec((8, C), lambda b: (0, 0))],
        out_specs=[pl.BlockSpec((TB, C), lambda b: (b, 0)),
                   pl.BlockSpec((8, TB), lambda b: (0, b))],
        out_shape=[jax.ShapeDtypeStruct((N, C), jnp.float32),
                   jax.ShapeDtypeStruct((8, N), jnp.float32)],
    )(mt, u, xw, dv, wpr, bb)


def _aggscore(mt, sw, dv, bp):
    bpb = jnp.broadcast_to(jnp.reshape(bp, (1, 1)), (8, 128))
    return pl.pallas_call(
        _aggscore_kernel,
        grid=(N // TB,),
        in_specs=[pl.BlockSpec((TB, N), lambda b: (b, 0)),
                  pl.BlockSpec((8, N), lambda b: (0, 0)),
                  pl.BlockSpec((8, N), lambda b: (0, 0)),
                  pl.BlockSpec((8, 128), lambda b: (0, 0))],
        out_specs=pl.BlockSpec((8, TB), lambda b: (0, b)),
        out_shape=jax.ShapeDtypeStruct((8, N), jnp.float32),
    )(mt, sw, dv, bpb)


def _aggdeg(mt, kp):
    return pl.pallas_call(
        _aggdeg_kernel,
        grid=(N // TB,),
        in_specs=[pl.BlockSpec((TB, N), lambda b: (b, 0)),
                  pl.BlockSpec((8, N), lambda b: (0, 0))],
        out_specs=pl.BlockSpec((8, TB), lambda b: (0, b)),
        out_shape=jax.ShapeDtypeStruct((8, N), jnp.float32),
    )(mt, kp)


def _readout(h, s, kp, k):
    mx, sm = pl.pallas_call(
        _readout_kernel,
        grid=(NT,),
        in_specs=[pl.BlockSpec((T, C), lambda i: (i, 0)),
                  pl.BlockSpec((8, N), lambda i: (0, 0)),
                  pl.BlockSpec((8, N), lambda i: (0, 0))],
        out_specs=[pl.BlockSpec((8, C), lambda i: (0, 0)),
                   pl.BlockSpec((8, C), lambda i: (0, 0))],
        out_shape=[jax.ShapeDtypeStruct((8, C), jnp.float32),
                   jax.ShapeDtypeStruct((8, C), jnp.float32)],
    )(h, s, kp)
    return jnp.concatenate([mx[0], sm[0] / k])


def _pool(sc, kx):
    tau = jax.lax.top_k(sc, kx)[0][kx - 1]
    keep_n = (sc >= tau).astype(jnp.float32)
    s_n = keep_n * jnp.tanh(sc)
    s8 = jnp.broadcast_to(s_n[None, :], (8, N))
    k8 = jnp.broadcast_to(keep_n[None, :], (8, N))
    return s8, k8


def kernel(feature, W1, b1, Wp1, bp1, W2, b2, Wp2, bp2, W3, b3, Wp3, bp3):
    f32 = jnp.float32
    fp = jnp.pad(feature.astype(f32), ((0, 0), (0, KF - feature.shape[1])))
    w1p = jnp.pad(W1, ((0, KF - W1.shape[0]), (0, 0)))

    d, mx = _dist(fp)
    t = 0.5 * mx[0, 0]
    mt, dv1 = _mask(d, t)

    ones8 = jnp.ones((8, N), f32)
    k1, k2, k3 = 3072, 2304, 1728

    xw1, u1, _, _ = _lin(fp, w1p, ones8, dv1, ones8)
    h1, sw1 = _aggconv(mt, u1, xw1, dv1, Wp1, b1)
    sc1 = _aggscore(mt, sw1, dv1, bp1)[0]
    s1, kp2 = _pool(sc1, k1)
    dv2 = _aggdeg(mt, kp2)

    xw2, u2, mx1, sm1 = _lin(h1, W2, s1, dv2, kp2)
    x1r = jnp.concatenate([mx1[0], sm1[0] / k1])
    h2, sw2 = _aggconv(mt, u2, xw2, dv2, Wp2, b2)
    sc2 = _aggscore(mt, sw2, dv2, bp2)[0]
    s2, kp3 = _pool(sc2, k2)
    dv3 = _aggdeg(mt, kp3)

    xw3, u3, mx2, sm2 = _lin(h2, W3, s2, dv3, kp3)
    x2r = jnp.concatenate([mx2[0], sm2[0] / k2])
    h3, sw3 = _aggconv(mt, u3, xw3, dv3, Wp3, b3)
    sc3 = _aggscore(mt, sw3, dv3, bp3)[0]
    s3, kp4 = _pool(sc3, k3)
    x3r = _readout(h3, s3, kp4, k3)

    return (x1r + x2r + x3r)[None, :]


# speedup vs baseline: 1.1066x; 1.0532x over previous
import jax
import jax.numpy as jnp
from jax.experimental import pallas as pl

N = 4096
T = 256
NT = N // T
JT = 1024
TB = 512
C = 256
KF = 512
NEG = -1e30


def _dist_kernel(f_ref, d_ref, mx_ref):
    i = pl.program_id(0)
    j = pl.program_id(1)
    a = f_ref[pl.ds(i * T, T), :]
    b = f_ref[pl.ds(j * JT, JT), :]
    xi2 = jnp.sum(a * a, axis=1)
    xj2 = jnp.sum(b * b, axis=1)
    g = jax.lax.dot_general(a, b, (((1,), (1,)), ((), ())),
                            preferred_element_type=jnp.float32)
    dist = xi2[:, None] + xj2[None, :] - 2.0 * g
    d_ref[...] = dist
    m = jnp.max(dist)
    first = (i == 0) & (j == 0)

    @pl.when(first)
    def _():
        mx_ref[...] = jnp.full((8, 128), m, jnp.float32)

    @pl.when(jnp.logical_not(first))
    def _():
        mx_ref[...] = jnp.maximum(mx_ref[...], m)


def _mask_kernel(t_ref, d_ref, mt_ref, dv_ref):
    b = pl.program_id(0)
    t = t_ref[0, 0]
    d = d_ref[...]
    gb = b * T + jax.lax.broadcasted_iota(jnp.int32, (T, N), 0)
    ga = jax.lax.broadcasted_iota(jnp.int32, (T, N), 1)
    mf = ((d < t) & (ga < gb)).astype(jnp.float32)
    mt_ref[...] = mf.astype(jnp.bfloat16)
    dv = jax.lax.rsqrt(1.0 + jnp.sum(mf, axis=1))
    dv_ref[...] = jnp.broadcast_to(dv[None, :], (8, T))


def _lin_kernel(s_ref, dv_ref, kp_ref, x_ref, w_ref, o_ref, u_ref,
                mx_ref, sm_ref):
    i = pl.program_id(0)
    s = s_ref[0, pl.ds(i * T, T)][:, None]
    dv = dv_ref[0, pl.ds(i * T, T)][:, None]
    xs = x_ref[...] * s
    xw = jnp.dot(xs, w_ref[...], preferred_element_type=jnp.float32)
    o_ref[...] = xw
    u_ref[...] = (xw * dv).astype(jnp.bfloat16)
    keep = kp_ref[0, pl.ds(i * T, T)][:, None] > 0.0
    tmax = jnp.max(jnp.where(keep, xs, NEG), axis=0)
    tsum = jnp.sum(jnp.where(keep, xs, 0.0), axis=0)

    @pl.when(i == 0)
    def _():
        mx_ref[...] = jnp.full_like(mx_ref, NEG)
        sm_ref[...] = jnp.zeros_like(sm_ref)

    mx_ref[...] = jnp.maximum(mx_ref[...], tmax[None, :])
    sm_ref[...] += tsum[None, :]


def _aggconv_kernel(mt_ref, u_ref, xwb_ref, dv_ref, wp_ref, b_ref,
                    o_ref, sw_ref):
    b = pl.program_id(0)
    acc = jnp.dot(mt_ref[...], u_ref[...], preferred_element_type=jnp.float32)
    dvb = dv_ref[0, pl.ds(b * TB, TB)][:, None]
    z = dvb * acc + dvb * dvb * xwb_ref[...] + b_ref[0, :][None, :]
    h = jnp.maximum(z, 0.0)
    o_ref[...] = h
    sw = jnp.sum(h * wp_ref[0, :][None, :], axis=1)
    sw_ref[...] = jnp.broadcast_to(sw[None, :], (8, TB))


def _aggscore_kernel(mt_ref, sw_ref, dv_ref, bp_ref, o_ref):
    b = pl.program_id(0)
    u = sw_ref[0, :] * dv_ref[0, :]
    acc = jnp.sum(mt_ref[...].astype(jnp.float32) * u[None, :], axis=1)
    dvb = dv_ref[0, pl.ds(b * TB, TB)]
    swb = sw_ref[0, pl.ds(b * TB, TB)]
    z = dvb * acc + dvb * dvb * swb + bp_ref[0, 0]
    z = jnp.where(dvb > 0.0, z, NEG)
    o_ref[...] = jnp.broadcast_to(z[None, :], (8, TB))


def _deglin_kernel(mt_ref, s_ref, kp_ref, x_ref, w_ref,
                   o_ref, u_ref, dv_ref, mx_ref, sm_ref):
    b = pl.program_id(0)
    kp = kp_ref[0, :]
    acc = jnp.sum(mt_ref[...].astype(jnp.float32) * kp[None, :], axis=1)
    dvb = kp_ref[0, pl.ds(b * TB, TB)] * jax.lax.rsqrt(1.0 + acc)
    dv_ref[...] = jnp.broadcast_to(dvb[None, :], (8, TB))
    s = s_ref[0, pl.ds(b * TB, TB)][:, None]
    xs = x_ref[...] * s
    xw = jnp.dot(xs, w_ref[...], preferred_element_type=jnp.float32)
    o_ref[...] = xw
    u_ref[...] = (xw * dvb[:, None]).astype(jnp.bfloat16)
    keep = kp_ref[0, pl.ds(b * TB, TB)][:, None] > 0.0
    tmax = jnp.max(jnp.where(keep, xs, NEG), axis=0)
    tsum = jnp.sum(jnp.where(keep, xs, 0.0), axis=0)

    @pl.when(b == 0)
    def _():
        mx_ref[...] = jnp.full_like(mx_ref, NEG)
        sm_ref[...] = jnp.zeros_like(sm_ref)

    mx_ref[...] = jnp.maximum(mx_ref[...], tmax[None, :])
    sm_ref[...] += tsum[None, :]


def _aggdeg_kernel(mt_ref, kp_ref, o_ref):
    b = pl.program_id(0)
    u = kp_ref[0, :]
    acc = jnp.sum(mt_ref[...].astype(jnp.float32) * u[None, :], axis=1)
    dv = kp_ref[0, pl.ds(b * TB, TB)] * jax.lax.rsqrt(1.0 + acc)
    o_ref[...] = jnp.broadcast_to(dv[None, :], (8, TB))


def _readout_kernel(h_ref, s_ref, k_ref, mx_ref, sm_ref):
    i = pl.program_id(0)
    s = s_ref[0, pl.ds(i * T, T)][:, None]
    keep = k_ref[0, pl.ds(i * T, T)][:, None] > 0.0
    x = h_ref[...] * s
    tmax = jnp.max(jnp.where(keep, x, NEG), axis=0)
    tsum = jnp.sum(jnp.where(keep, x, 0.0), axis=0)

    @pl.when(i == 0)
    def _():
        mx_ref[...] = jnp.full_like(mx_ref, NEG)
        sm_ref[...] = jnp.zeros_like(sm_ref)

    mx_ref[...] = jnp.maximum(mx_ref[...], tmax[None, :])
    sm_ref[...] += tsum[None, :]


def _dist(fp):
    return pl.pallas_call(
        _dist_kernel,
        grid=(NT, N // JT),
        in_specs=[pl.BlockSpec((N, KF), lambda i, j: (0, 0))],
        out_specs=[pl.BlockSpec((T, JT), lambda i, j: (i, j)),
                   pl.BlockSpec((8, 128), lambda i, j: (0, 0))],
        out_shape=[jax.ShapeDtypeStruct((N, N), jnp.float32),
                   jax.ShapeDtypeStruct((8, 128), jnp.float32)],
    )(fp)


def _mask(d, t):
    tb = jnp.broadcast_to(jnp.reshape(t, (1, 1)), (8, 128))
    return pl.pallas_call(
        _mask_kernel,
        grid=(NT,),
        in_specs=[pl.BlockSpec((8, 128), lambda b: (0, 0)),
                  pl.BlockSpec((T, N), lambda b: (b, 0))],
        out_specs=[pl.BlockSpec((T, N), lambda b: (b, 0)),
                   pl.BlockSpec((8, T), lambda b: (0, b))],
        out_shape=[jax.ShapeDtypeStruct((N, N), jnp.bfloat16),
                   jax.ShapeDtypeStruct((8, N), jnp.float32)],
    )(tb, d)


def _lin(x, w, s, dv, kp):
    kd, m = w.shape
    return pl.pallas_call(
        _lin_kernel,
        grid=(NT,),
        in_specs=[pl.BlockSpec((8, N), lambda i: (0, 0)),
                  pl.BlockSpec((8, N), lambda i: (0, 0)),
                  pl.BlockSpec((8, N), lambda i: (0, 0)),
                  pl.BlockSpec((T, kd), lambda i: (i, 0)),
                  pl.BlockSpec((kd, m), lambda i: (0, 0))],
        out_specs=[pl.BlockSpec((T, m), lambda i: (i, 0)),
                   pl.BlockSpec((T, m), lambda i: (i, 0)),
                   pl.BlockSpec((8, kd), lambda i: (0, 0)),
                   pl.BlockSpec((8, kd), lambda i: (0, 0))],
        out_shape=[jax.ShapeDtypeStruct((N, m), jnp.float32),
                   jax.ShapeDtypeStruct((N, m), jnp.bfloat16),
                   jax.ShapeDtypeStruct((8, kd), jnp.float32),
                   jax.ShapeDtypeStruct((8, kd), jnp.float32)],
    )(s, dv, kp, x, w)


def _aggconv(mt, u, xw, dv, wp, bias):
    wpr = jnp.broadcast_to(wp[:, 0][None, :], (8, C))
    bb = jnp.broadcast_to(bias[None, :], (8, C))
    return pl.pallas_call(
        _aggconv_kernel,
        grid=(N // TB,),
        in_specs=[pl.BlockSpec((TB, N), lambda b: (b, 0)),
                  pl.BlockSpec((N, C), lambda b: (0, 0)),
                  pl.BlockSpec((TB, C), lambda b: (b, 0)),
                  pl.BlockSpec((8, N), lambda b: (0, 0)),
                  pl.BlockSpec((8, C), lambda b: (0, 0)),
                  pl.BlockSpec((8, C), lambda b: (0, 0))],
        out_specs=[pl.BlockSpec((TB, C), lambda b: (b, 0)),
                   pl.BlockSpec((8, TB), lambda b: (0, b))],
        out_shape=[jax.ShapeDtypeStruct((N, C), jnp.float32),
                   jax.ShapeDtypeStruct((8, N), jnp.float32)],
    )(mt, u, xw, dv, wpr, bb)


def _aggscore(mt, sw, dv, bp):
    bpb = jnp.broadcast_to(jnp.reshape(bp, (1, 1)), (8, 128))
    return pl.pallas_call(
        _aggscore_kernel,
        grid=(N // TB,),
        in_specs=[pl.BlockSpec((TB, N), lambda b: (b, 0)),
                  pl.BlockSpec((8, N), lambda b: (0, 0)),
                  pl.BlockSpec((8, N), lambda b: (0, 0)),
                  pl.BlockSpec((8, 128), lambda b: (0, 0))],
        out_specs=pl.BlockSpec((8, TB), lambda b: (0, b)),
        out_shape=jax.ShapeDtypeStruct((8, N), jnp.float32),
    )(mt, sw, dv, bpb)


def _deglin(mt, x, w, s, kp):
    kd, m = w.shape
    return pl.pallas_call(
        _deglin_kernel,
        grid=(N // TB,),
        in_specs=[pl.BlockSpec((TB, N), lambda b: (b, 0)),
                  pl.BlockSpec((8, N), lambda b: (0, 0)),
                  pl.BlockSpec((8, N), lambda b: (0, 0)),
                  pl.BlockSpec((TB, kd), lambda b: (b, 0)),
                  pl.BlockSpec((kd, m), lambda b: (0, 0))],
        out_specs=[pl.BlockSpec((TB, m), lambda b: (b, 0)),
                   pl.BlockSpec((TB, m), lambda b: (b, 0)),
                   pl.BlockSpec((8, TB), lambda b: (0, b)),
                   pl.BlockSpec((8, kd), lambda b: (0, 0)),
                   pl.BlockSpec((8, kd), lambda b: (0, 0))],
        out_shape=[jax.ShapeDtypeStruct((N, m), jnp.float32),
                   jax.ShapeDtypeStruct((N, m), jnp.bfloat16),
                   jax.ShapeDtypeStruct((8, N), jnp.float32),
                   jax.ShapeDtypeStruct((8, kd), jnp.float32),
                   jax.ShapeDtypeStruct((8, kd), jnp.float32)],
    )(mt, s, kp, x, w)


def _aggdeg(mt, kp):
    return pl.pallas_call(
        _aggdeg_kernel,
        grid=(N // TB,),
        in_specs=[pl.BlockSpec((TB, N), lambda b: (b, 0)),
                  pl.BlockSpec((8, N), lambda b: (0, 0))],
        out_specs=pl.BlockSpec((8, TB), lambda b: (0, b)),
        out_shape=jax.ShapeDtypeStruct((8, N), jnp.float32),
    )(mt, kp)


def _readout(h, s, kp, k):
    mx, sm = pl.pallas_call(
        _readout_kernel,
        grid=(NT,),
        in_specs=[pl.BlockSpec((T, C), lambda i: (i, 0)),
                  pl.BlockSpec((8, N), lambda i: (0, 0)),
                  pl.BlockSpec((8, N), lambda i: (0, 0))],
        out_specs=[pl.BlockSpec((8, C), lambda i: (0, 0)),
                   pl.BlockSpec((8, C), lambda i: (0, 0))],
        out_shape=[jax.ShapeDtypeStruct((8, C), jnp.float32),
                   jax.ShapeDtypeStruct((8, C), jnp.float32)],
    )(h, s, kp)
    return jnp.concatenate([mx[0], sm[0] / k])


def _pool(sc, kx):
    tau = jax.lax.top_k(sc, kx)[0][kx - 1]
    keep_n = (sc >= tau).astype(jnp.float32)
    s_n = keep_n * jnp.tanh(sc)
    s8 = jnp.broadcast_to(s_n[None, :], (8, N))
    k8 = jnp.broadcast_to(keep_n[None, :], (8, N))
    return s8, k8


def kernel(feature, W1, b1, Wp1, bp1, W2, b2, Wp2, bp2, W3, b3, Wp3, bp3):
    f32 = jnp.float32
    fp = jnp.pad(feature.astype(f32), ((0, 0), (0, KF - feature.shape[1])))
    w1p = jnp.pad(W1, ((0, KF - W1.shape[0]), (0, 0)))

    d, mx = _dist(fp)
    t = 0.5 * mx[0, 0]
    mt, dv1 = _mask(d, t)

    ones8 = jnp.ones((8, N), f32)
    k1, k2, k3 = 3072, 2304, 1728

    xw1, u1, _, _ = _lin(fp, w1p, ones8, dv1, ones8)
    h1, sw1 = _aggconv(mt, u1, xw1, dv1, Wp1, b1)
    sc1 = _aggscore(mt, sw1, dv1, bp1)[0]
    s1, kp2 = _pool(sc1, k1)

    xw2, u2, dv2, mx1, sm1 = _deglin(mt, h1, W2, s1, kp2)
    x1r = jnp.concatenate([mx1[0], sm1[0] / k1])
    h2, sw2 = _aggconv(mt, u2, xw2, dv2, Wp2, b2)
    sc2 = _aggscore(mt, sw2, dv2, bp2)[0]
    s2, kp3 = _pool(sc2, k2)

    xw3, u3, dv3, mx2, sm2 = _deglin(mt, h2, W3, s2, kp3)
    x2r = jnp.concatenate([mx2[0], sm2[0] / k2])
    h3, sw3 = _aggconv(mt, u3, xw3, dv3, Wp3, b3)
    sc3 = _aggscore(mt, sw3, dv3, bp3)[0]
    s3, kp4 = _pool(sc3, k3)
    x3r = _readout(h3, s3, kp4, k3)

    return (x1r + x2r + x3r)[None, :]


# speedup vs baseline: 1.1532x; 1.0421x over previous
import jax
import jax.numpy as jnp
from jax.experimental import pallas as pl

N = 4096
T = 256
NT = N // T
JT = 2048
TB = 512
C = 256
KF = 512
NEG = -1e30


def _dist_kernel(f_ref, d_ref, mx_ref):
    i = pl.program_id(0)
    j = pl.program_id(1)
    a = f_ref[pl.ds(i * T, T), :]
    b = f_ref[pl.ds(j * JT, JT), :]
    xi2 = jnp.sum(a * a, axis=1)
    xj2 = jnp.sum(b * b, axis=1)
    g = jax.lax.dot_general(a, b, (((1,), (1,)), ((), ())),
                            preferred_element_type=jnp.float32)
    dist = xi2[:, None] + xj2[None, :] - 2.0 * g
    d_ref[...] = dist
    m = jnp.max(dist)
    first = (i == 0) & (j == 0)

    @pl.when(first)
    def _():
        mx_ref[...] = jnp.full((8, 128), m, jnp.float32)

    @pl.when(jnp.logical_not(first))
    def _():
        mx_ref[...] = jnp.maximum(mx_ref[...], m)


def _mask_kernel(t_ref, d_ref, mt_ref, dv_ref):
    b = pl.program_id(0)
    t = t_ref[0, 0]
    d = d_ref[...]
    gb = b * TB + jax.lax.broadcasted_iota(jnp.int32, (TB, N), 0)
    ga = jax.lax.broadcasted_iota(jnp.int32, (TB, N), 1)
    mf = ((d < t) & (ga < gb)).astype(jnp.float32)
    mt_ref[...] = mf.astype(jnp.bfloat16)
    dv = jax.lax.rsqrt(1.0 + jnp.sum(mf, axis=1))
    dv_ref[...] = jnp.broadcast_to(dv[None, :], (8, TB))


def _lin_kernel(s_ref, dv_ref, kp_ref, x_ref, w_ref, o_ref, u_ref,
                mx_ref, sm_ref):
    i = pl.program_id(0)
    s = s_ref[0, pl.ds(i * T, T)][:, None]
    dv = dv_ref[0, pl.ds(i * T, T)][:, None]
    xs = x_ref[...] * s
    xw = jnp.dot(xs, w_ref[...], preferred_element_type=jnp.float32)
    o_ref[...] = xw
    u_ref[...] = (xw * dv).astype(jnp.bfloat16)
    keep = kp_ref[0, pl.ds(i * T, T)][:, None] > 0.0
    tmax = jnp.max(jnp.where(keep, xs, NEG), axis=0)
    tsum = jnp.sum(jnp.where(keep, xs, 0.0), axis=0)

    @pl.when(i == 0)
    def _():
        mx_ref[...] = jnp.full_like(mx_ref, NEG)
        sm_ref[...] = jnp.zeros_like(sm_ref)

    mx_ref[...] = jnp.maximum(mx_ref[...], tmax[None, :])
    sm_ref[...] += tsum[None, :]


def _aggconv_kernel(mt_ref, u_ref, xwb_ref, dv_ref, wp_ref, b_ref,
                    o_ref, sw_ref):
    b = pl.program_id(0)
    acc = jnp.dot(mt_ref[...], u_ref[...], preferred_element_type=jnp.float32)
    dvb = dv_ref[0, pl.ds(b * TB, TB)][:, None]
    z = dvb * acc + dvb * dvb * xwb_ref[...] + b_ref[0, :][None, :]
    h = jnp.maximum(z, 0.0)
    o_ref[...] = h
    sw = jnp.sum(h * wp_ref[0, :][None, :], axis=1)
    sw_ref[...] = jnp.broadcast_to(sw[None, :], (8, TB))


def _aggscore_kernel(mt_ref, sw_ref, dv_ref, bp_ref, o_ref):
    b = pl.program_id(0)
    u = sw_ref[0, :] * dv_ref[0, :]
    acc = jnp.sum(mt_ref[...].astype(jnp.float32) * u[None, :], axis=1)
    dvb = dv_ref[0, pl.ds(b * TB, TB)]
    swb = sw_ref[0, pl.ds(b * TB, TB)]
    z = dvb * acc + dvb * dvb * swb + bp_ref[0, 0]
    z = jnp.where(dvb > 0.0, z, NEG)
    o_ref[...] = jnp.broadcast_to(z[None, :], (8, TB))


def _deglin_kernel(mt_ref, s_ref, kp_ref, x_ref, w_ref,
                   o_ref, u_ref, dv_ref, mx_ref, sm_ref):
    b = pl.program_id(0)
    kp = kp_ref[0, :]
    acc = jnp.sum(mt_ref[...].astype(jnp.float32) * kp[None, :], axis=1)
    dvb = kp_ref[0, pl.ds(b * TB, TB)] * jax.lax.rsqrt(1.0 + acc)
    dv_ref[...] = jnp.broadcast_to(dvb[None, :], (8, TB))
    s = s_ref[0, pl.ds(b * TB, TB)][:, None]
    xs = x_ref[...] * s
    xw = jnp.dot(xs, w_ref[...], preferred_element_type=jnp.float32)
    o_ref[...] = xw
    u_ref[...] = (xw * dvb[:, None]).astype(jnp.bfloat16)
    keep = kp_ref[0, pl.ds(b * TB, TB)][:, None] > 0.0
    tmax = jnp.max(jnp.where(keep, xs, NEG), axis=0)
    tsum = jnp.sum(jnp.where(keep, xs, 0.0), axis=0)

    @pl.when(b == 0)
    def _():
        mx_ref[...] = jnp.full_like(mx_ref, NEG)
        sm_ref[...] = jnp.zeros_like(sm_ref)

    mx_ref[...] = jnp.maximum(mx_ref[...], tmax[None, :])
    sm_ref[...] += tsum[None, :]


def _aggdeg_kernel(mt_ref, kp_ref, o_ref):
    b = pl.program_id(0)
    u = kp_ref[0, :]
    acc = jnp.sum(mt_ref[...].astype(jnp.float32) * u[None, :], axis=1)
    dv = kp_ref[0, pl.ds(b * TB, TB)] * jax.lax.rsqrt(1.0 + acc)
    o_ref[...] = jnp.broadcast_to(dv[None, :], (8, TB))


def _readout_kernel(h_ref, s_ref, k_ref, mx_ref, sm_ref):
    i = pl.program_id(0)
    s = s_ref[0, pl.ds(i * T, T)][:, None]
    keep = k_ref[0, pl.ds(i * T, T)][:, None] > 0.0
    x = h_ref[...] * s
    tmax = jnp.max(jnp.where(keep, x, NEG), axis=0)
    tsum = jnp.sum(jnp.where(keep, x, 0.0), axis=0)

    @pl.when(i == 0)
    def _():
        mx_ref[...] = jnp.full_like(mx_ref, NEG)
        sm_ref[...] = jnp.zeros_like(sm_ref)

    mx_ref[...] = jnp.maximum(mx_ref[...], tmax[None, :])
    sm_ref[...] += tsum[None, :]


def _dist(fp):
    return pl.pallas_call(
        _dist_kernel,
        grid=(NT, N // JT),
        in_specs=[pl.BlockSpec((N, KF), lambda i, j: (0, 0))],
        out_specs=[pl.BlockSpec((T, JT), lambda i, j: (i, j)),
                   pl.BlockSpec((8, 128), lambda i, j: (0, 0))],
        out_shape=[jax.ShapeDtypeStruct((N, N), jnp.float32),
                   jax.ShapeDtypeStruct((8, 128), jnp.float32)],
    )(fp)


def _mask(d, t):
    tb = jnp.broadcast_to(jnp.reshape(t, (1, 1)), (8, 128))
    return pl.pallas_call(
        _mask_kernel,
        grid=(N // TB,),
        in_specs=[pl.BlockSpec((8, 128), lambda b: (0, 0)),
                  pl.BlockSpec((TB, N), lambda b: (b, 0))],
        out_specs=[pl.BlockSpec((TB, N), lambda b: (b, 0)),
                   pl.BlockSpec((8, TB), lambda b: (0, b))],
        out_shape=[jax.ShapeDtypeStruct((N, N), jnp.bfloat16),
                   jax.ShapeDtypeStruct((8, N), jnp.float32)],
    )(tb, d)


def _lin(x, w, s, dv, kp):
    kd, m = w.shape
    return pl.pallas_call(
        _lin_kernel,
        grid=(NT,),
        in_specs=[pl.BlockSpec((8, N), lambda i: (0, 0)),
                  pl.BlockSpec((8, N), lambda i: (0, 0)),
                  pl.BlockSpec((8, N), lambda i: (0, 0)),
                  pl.BlockSpec((T, kd), lambda i: (i, 0)),
                  pl.BlockSpec((kd, m), lambda i: (0, 0))],
        out_specs=[pl.BlockSpec((T, m), lambda i: (i, 0)),
                   pl.BlockSpec((T, m), lambda i: (i, 0)),
                   pl.BlockSpec((8, kd), lambda i: (0, 0)),
                   pl.BlockSpec((8, kd), lambda i: (0, 0))],
        out_shape=[jax.ShapeDtypeStruct((N, m), jnp.float32),
                   jax.ShapeDtypeStruct((N, m), jnp.bfloat16),
                   jax.ShapeDtypeStruct((8, kd), jnp.float32),
                   jax.ShapeDtypeStruct((8, kd), jnp.float32)],
    )(s, dv, kp, x, w)


def _aggconv(mt, u, xw, dv, wp, bias):
    wpr = jnp.broadcast_to(wp[:, 0][None, :], (8, C))
    bb = jnp.broadcast_to(bias[None, :], (8, C))
    return pl.pallas_call(
        _aggconv_kernel,
        grid=(N // TB,),
        in_specs=[pl.BlockSpec((TB, N), lambda b: (b, 0)),
                  pl.BlockSpec((N, C), lambda b: (0, 0)),
                  pl.BlockSpec((TB, C), lambda b: (b, 0)),
                  pl.BlockSpec((8, N), lambda b: (0, 0)),
                  pl.BlockSpec((8, C), lambda b: (0, 0)),
                  pl.BlockSpec((8, C), lambda b: (0, 0))],
        out_specs=[pl.BlockSpec((TB, C), lambda b: (b, 0)),
                   pl.BlockSpec((8, TB), lambda b: (0, b))],
        out_shape=[jax.ShapeDtypeStruct((N, C), jnp.float32),
                   jax.ShapeDtypeStruct((8, N), jnp.float32)],
    )(mt, u, xw, dv, wpr, bb)


def _aggscore(mt, sw, dv, bp):
    bpb = jnp.broadcast_to(jnp.reshape(bp, (1, 1)), (8, 128))
    return pl.pallas_call(
        _aggscore_kernel,
        grid=(N // TB,),
        in_specs=[pl.BlockSpec((TB, N), lambda b: (b, 0)),
                  pl.BlockSpec((8, N), lambda b: (0, 0)),
                  pl.BlockSpec((8, N), lambda b: (0, 0)),
                  pl.BlockSpec((8, 128), lambda b: (0, 0))],
        out_specs=pl.BlockSpec((8, TB), lambda b: (0, b)),
        out_shape=jax.ShapeDtypeStruct((8, N), jnp.float32),
    )(mt, sw, dv, bpb)


def _deglin(mt, x, w, s, kp):
    kd, m = w.shape
    return pl.pallas_call(
        _deglin_kernel,
        grid=(N // TB,),
        in_specs=[pl.BlockSpec((TB, N), lambda b: (b, 0)),
                  pl.BlockSpec((8, N), lambda b: (0, 0)),
                  pl.BlockSpec((8, N), lambda b: (0, 0)),
                  pl.BlockSpec((TB, kd), lambda b: (b, 0)),
                  pl.BlockSpec((kd, m), lambda b: (0, 0))],
        out_specs=[pl.BlockSpec((TB, m), lambda b: (b, 0)),
                   pl.BlockSpec((TB, m), lambda b: (b, 0)),
                   pl.BlockSpec((8, TB), lambda b: (0, b)),
                   pl.BlockSpec((8, kd), lambda b: (0, 0)),
                   pl.BlockSpec((8, kd), lambda b: (0, 0))],
        out_shape=[jax.ShapeDtypeStruct((N, m), jnp.float32),
                   jax.ShapeDtypeStruct((N, m), jnp.bfloat16),
                   jax.ShapeDtypeStruct((8, N), jnp.float32),
                   jax.ShapeDtypeStruct((8, kd), jnp.float32),
                   jax.ShapeDtypeStruct((8, kd), jnp.float32)],
    )(mt, s, kp, x, w)


def _aggdeg(mt, kp):
    return pl.pallas_call(
        _aggdeg_kernel,
        grid=(N // TB,),
        in_specs=[pl.BlockSpec((TB, N), lambda b: (b, 0)),
                  pl.BlockSpec((8, N), lambda b: (0, 0))],
        out_specs=pl.BlockSpec((8, TB), lambda b: (0, b)),
        out_shape=jax.ShapeDtypeStruct((8, N), jnp.float32),
    )(mt, kp)


def _readout(h, s, kp, k):
    mx, sm = pl.pallas_call(
        _readout_kernel,
        grid=(NT,),
        in_specs=[pl.BlockSpec((T, C), lambda i: (i, 0)),
                  pl.BlockSpec((8, N), lambda i: (0, 0)),
                  pl.BlockSpec((8, N), lambda i: (0, 0))],
        out_specs=[pl.BlockSpec((8, C), lambda i: (0, 0)),
                   pl.BlockSpec((8, C), lambda i: (0, 0))],
        out_shape=[jax.ShapeDtypeStruct((8, C), jnp.float32),
                   jax.ShapeDtypeStruct((8, C), jnp.float32)],
    )(h, s, kp)
    return jnp.concatenate([mx[0], sm[0] / k])


def _pool(sc, kx):
    tau = jax.lax.top_k(sc, kx)[0][kx - 1]
    keep_n = (sc >= tau).astype(jnp.float32)
    s_n = keep_n * jnp.tanh(sc)
    s8 = jnp.broadcast_to(s_n[None, :], (8, N))
    k8 = jnp.broadcast_to(keep_n[None, :], (8, N))
    return s8, k8


def kernel(feature, W1, b1, Wp1, bp1, W2, b2, Wp2, bp2, W3, b3, Wp3, bp3):
    f32 = jnp.float32
    fp = jnp.pad(feature.astype(f32), ((0, 0), (0, KF - feature.shape[1])))
    w1p = jnp.pad(W1, ((0, KF - W1.shape[0]), (0, 0)))

    d, mx = _dist(fp)
    t = 0.5 * mx[0, 0]
    mt, dv1 = _mask(d, t)

    ones8 = jnp.ones((8, N), f32)
    k1, k2, k3 = 3072, 2304, 1728

    xw1, u1, _, _ = _lin(fp, w1p, ones8, dv1, ones8)
    h1, sw1 = _aggconv(mt, u1, xw1, dv1, Wp1, b1)
    sc1 = _aggscore(mt, sw1, dv1, bp1)[0]
    s1, kp2 = _pool(sc1, k1)

    xw2, u2, dv2, mx1, sm1 = _deglin(mt, h1, W2, s1, kp2)
    x1r = jnp.concatenate([mx1[0], sm1[0] / k1])
    h2, sw2 = _aggconv(mt, u2, xw2, dv2, Wp2, b2)
    sc2 = _aggscore(mt, sw2, dv2, bp2)[0]
    s2, kp3 = _pool(sc2, k2)

    xw3, u3, dv3, mx2, sm2 = _deglin(mt, h2, W3, s2, kp3)
    x2r = jnp.concatenate([mx2[0], sm2[0] / k2])
    h3, sw3 = _aggconv(mt, u3, xw3, dv3, Wp3, b3)
    sc3 = _aggscore(mt, sw3, dv3, bp3)[0]
    s3, kp4 = _pool(sc3, k3)
    x3r = _readout(h3, s3, kp4, k3)

    return (x1r + x2r + x3r)[None, :]


# speedup vs baseline: 1.1944x; 1.0357x over previous
import jax
import jax.numpy as jnp
from jax.experimental import pallas as pl

N = 4096
T = 256
NT = N // T
JT = 2048
TB = 512
C = 256
KF = 512
NEG = -1e30


def _dist_kernel(f_ref, d_ref, mx_ref):
    i = pl.program_id(0)
    j = pl.program_id(1)
    a = f_ref[pl.ds(i * TB, TB), :]
    b = f_ref[pl.ds(j * JT, JT), :]
    xi2 = jnp.sum(a * a, axis=1)
    xj2 = jnp.sum(b * b, axis=1)
    g = jax.lax.dot_general(a, b, (((1,), (1,)), ((), ())),
                            preferred_element_type=jnp.float32)
    dist = xi2[:, None] + xj2[None, :] - 2.0 * g
    d_ref[...] = dist
    m = jnp.max(dist)
    first = (i == 0) & (j == 0)

    @pl.when(first)
    def _():
        mx_ref[...] = jnp.full((8, 128), m, jnp.float32)

    @pl.when(jnp.logical_not(first))
    def _():
        mx_ref[...] = jnp.maximum(mx_ref[...], m)


def _mask_kernel(t_ref, d_ref, mt_ref, dv_ref):
    b = pl.program_id(0)
    t = t_ref[0, 0]
    d = d_ref[...]
    gb = b * TB + jax.lax.broadcasted_iota(jnp.int32, (TB, N), 0)
    ga = jax.lax.broadcasted_iota(jnp.int32, (TB, N), 1)
    mf = ((d < t) & (ga < gb)).astype(jnp.float32)
    mt_ref[...] = mf.astype(jnp.bfloat16)
    dv = jax.lax.rsqrt(1.0 + jnp.sum(mf, axis=1))
    dv_ref[...] = jnp.broadcast_to(dv[None, :], (8, TB))


def _lin_kernel(s_ref, dv_ref, kp_ref, x_ref, w_ref, o_ref, u_ref,
                mx_ref, sm_ref):
    i = pl.program_id(0)
    s = s_ref[0, pl.ds(i * T, T)][:, None]
    dv = dv_ref[0, pl.ds(i * T, T)][:, None]
    xs = x_ref[...] * s
    xw = jnp.dot(xs, w_ref[...], preferred_element_type=jnp.float32)
    o_ref[...] = xw
    u_ref[...] = (xw * dv).astype(jnp.bfloat16)
    keep = kp_ref[0, pl.ds(i * T, T)][:, None] > 0.0
    tmax = jnp.max(jnp.where(keep, xs, NEG), axis=0)
    tsum = jnp.sum(jnp.where(keep, xs, 0.0), axis=0)

    @pl.when(i == 0)
    def _():
        mx_ref[...] = jnp.full_like(mx_ref, NEG)
        sm_ref[...] = jnp.zeros_like(sm_ref)

    mx_ref[...] = jnp.maximum(mx_ref[...], tmax[None, :])
    sm_ref[...] += tsum[None, :]


def _aggconv_kernel(mt_ref, u_ref, xwb_ref, dv_ref, wp_ref, b_ref,
                    o_ref, sw_ref):
    b = pl.program_id(0)
    acc = jnp.dot(mt_ref[...], u_ref[...], preferred_element_type=jnp.float32)
    dvb = dv_ref[0, pl.ds(b * TB, TB)][:, None]
    z = dvb * acc + dvb * dvb * xwb_ref[...] + b_ref[0, :][None, :]
    h = jnp.maximum(z, 0.0)
    o_ref[...] = h
    sw = jnp.sum(h * wp_ref[0, :][None, :], axis=1)
    sw_ref[...] = jnp.broadcast_to(sw[None, :], (8, TB))


def _aggscore_kernel(mt_ref, sw_ref, dv_ref, bp_ref, o_ref):
    b = pl.program_id(0)
    u = sw_ref[0, :] * dv_ref[0, :]
    acc = jnp.sum(mt_ref[...].astype(jnp.float32) * u[None, :], axis=1)
    dvb = dv_ref[0, pl.ds(b * TB, TB)]
    swb = sw_ref[0, pl.ds(b * TB, TB)]
    z = dvb * acc + dvb * dvb * swb + bp_ref[0, 0]
    z = jnp.where(dvb > 0.0, z, NEG)
    o_ref[...] = jnp.broadcast_to(z[None, :], (8, TB))


def _deglin_kernel(mt_ref, s_ref, kp_ref, x_ref, w_ref,
                   o_ref, u_ref, dv_ref, mx_ref, sm_ref):
    b = pl.program_id(0)
    kp = kp_ref[0, :]
    acc = jnp.sum(mt_ref[...].astype(jnp.float32) * kp[None, :], axis=1)
    dvb = kp_ref[0, pl.ds(b * TB, TB)] * jax.lax.rsqrt(1.0 + acc)
    dv_ref[...] = jnp.broadcast_to(dvb[None, :], (8, TB))
    s = s_ref[0, pl.ds(b * TB, TB)][:, None]
    xs = x_ref[...] * s
    xw = jnp.dot(xs, w_ref[...], preferred_element_type=jnp.float32)
    o_ref[...] = xw
    u_ref[...] = (xw * dvb[:, None]).astype(jnp.bfloat16)
    keep = kp_ref[0, pl.ds(b * TB, TB)][:, None] > 0.0
    tmax = jnp.max(jnp.where(keep, xs, NEG), axis=0)
    tsum = jnp.sum(jnp.where(keep, xs, 0.0), axis=0)

    @pl.when(b == 0)
    def _():
        mx_ref[...] = jnp.full_like(mx_ref, NEG)
        sm_ref[...] = jnp.zeros_like(sm_ref)

    mx_ref[...] = jnp.maximum(mx_ref[...], tmax[None, :])
    sm_ref[...] += tsum[None, :]


def _aggdeg_kernel(mt_ref, kp_ref, o_ref):
    b = pl.program_id(0)
    u = kp_ref[0, :]
    acc = jnp.sum(mt_ref[...].astype(jnp.float32) * u[None, :], axis=1)
    dv = kp_ref[0, pl.ds(b * TB, TB)] * jax.lax.rsqrt(1.0 + acc)
    o_ref[...] = jnp.broadcast_to(dv[None, :], (8, TB))


def _readout_kernel(h_ref, s_ref, k_ref, mx_ref, sm_ref):
    i = pl.program_id(0)
    s = s_ref[0, pl.ds(i * T, T)][:, None]
    keep = k_ref[0, pl.ds(i * T, T)][:, None] > 0.0
    x = h_ref[...] * s
    tmax = jnp.max(jnp.where(keep, x, NEG), axis=0)
    tsum = jnp.sum(jnp.where(keep, x, 0.0), axis=0)

    @pl.when(i == 0)
    def _():
        mx_ref[...] = jnp.full_like(mx_ref, NEG)
        sm_ref[...] = jnp.zeros_like(sm_ref)

    mx_ref[...] = jnp.maximum(mx_ref[...], tmax[None, :])
    sm_ref[...] += tsum[None, :]


def _dist(fp):
    return pl.pallas_call(
        _dist_kernel,
        grid=(N // TB, N // JT),
        in_specs=[pl.BlockSpec((N, KF), lambda i, j: (0, 0))],
        out_specs=[pl.BlockSpec((TB, JT), lambda i, j: (i, j)),
                   pl.BlockSpec((8, 128), lambda i, j: (0, 0))],
        out_shape=[jax.ShapeDtypeStruct((N, N), jnp.float32),
                   jax.ShapeDtypeStruct((8, 128), jnp.float32)],
    )(fp)


def _mask(d, t):
    tb = jnp.broadcast_to(jnp.reshape(t, (1, 1)), (8, 128))
    return pl.pallas_call(
        _mask_kernel,
        grid=(N // TB,),
        in_specs=[pl.BlockSpec((8, 128), lambda b: (0, 0)),
                  pl.BlockSpec((TB, N), lambda b: (b, 0))],
        out_specs=[pl.BlockSpec((TB, N), lambda b: (b, 0)),
                   pl.BlockSpec((8, TB), lambda b: (0, b))],
        out_shape=[jax.ShapeDtypeStruct((N, N), jnp.bfloat16),
                   jax.ShapeDtypeStruct((8, N), jnp.float32)],
    )(tb, d)


def _lin(x, w, s, dv, kp):
    kd, m = w.shape
    return pl.pallas_call(
        _lin_kernel,
        grid=(NT,),
        in_specs=[pl.BlockSpec((8, N), lambda i: (0, 0)),
                  pl.BlockSpec((8, N), lambda i: (0, 0)),
                  pl.BlockSpec((8, N), lambda i: (0, 0)),
                  pl.BlockSpec((T, kd), lambda i: (i, 0)),
                  pl.BlockSpec((kd, m), lambda i: (0, 0))],
        out_specs=[pl.BlockSpec((T, m), lambda i: (i, 0)),
                   pl.BlockSpec((T, m), lambda i: (i, 0)),
                   pl.BlockSpec((8, kd), lambda i: (0, 0)),
                   pl.BlockSpec((8, kd), lambda i: (0, 0))],
        out_shape=[jax.ShapeDtypeStruct((N, m), jnp.float32),
                   jax.ShapeDtypeStruct((N, m), jnp.bfloat16),
                   jax.ShapeDtypeStruct((8, kd), jnp.float32),
                   jax.ShapeDtypeStruct((8, kd), jnp.float32)],
    )(s, dv, kp, x, w)


def _aggconv(mt, u, xw, dv, wp, bias):
    wpr = jnp.broadcast_to(wp[:, 0][None, :], (8, C))
    bb = jnp.broadcast_to(bias[None, :], (8, C))
    return pl.pallas_call(
        _aggconv_kernel,
        grid=(N // TB,),
        in_specs=[pl.BlockSpec((TB, N), lambda b: (b, 0)),
                  pl.BlockSpec((N, C), lambda b: (0, 0)),
                  pl.BlockSpec((TB, C), lambda b: (b, 0)),
                  pl.BlockSpec((8, N), lambda b: (0, 0)),
                  pl.BlockSpec((8, C), lambda b: (0, 0)),
                  pl.BlockSpec((8, C), lambda b: (0, 0))],
        out_specs=[pl.BlockSpec((TB, C), lambda b: (b, 0)),
                   pl.BlockSpec((8, TB), lambda b: (0, b))],
        out_shape=[jax.ShapeDtypeStruct((N, C), jnp.float32),
                   jax.ShapeDtypeStruct((8, N), jnp.float32)],
    )(mt, u, xw, dv, wpr, bb)


def _aggscore(mt, sw, dv, bp):
    bpb = jnp.broadcast_to(jnp.reshape(bp, (1, 1)), (8, 128))
    return pl.pallas_call(
        _aggscore_kernel,
        grid=(N // TB,),
        in_specs=[pl.BlockSpec((TB, N), lambda b: (b, 0)),
                  pl.BlockSpec((8, N), lambda b: (0, 0)),
                  pl.BlockSpec((8, N), lambda b: (0, 0)),
                  pl.BlockSpec((8, 128), lambda b: (0, 0))],
        out_specs=pl.BlockSpec((8, TB), lambda b: (0, b)),
        out_shape=jax.ShapeDtypeStruct((8, N), jnp.float32),
    )(mt, sw, dv, bpb)


def _deglin(mt, x, w, s, kp):
    kd, m = w.shape
    return pl.pallas_call(
        _deglin_kernel,
        grid=(N // TB,),
        in_specs=[pl.BlockSpec((TB, N), lambda b: (b, 0)),
                  pl.BlockSpec((8, N), lambda b: (0, 0)),
                  pl.BlockSpec((8, N), lambda b: (0, 0)),
                  pl.BlockSpec((TB, kd), lambda b: (b, 0)),
                  pl.BlockSpec((kd, m), lambda b: (0, 0))],
        out_specs=[pl.BlockSpec((TB, m), lambda b: (b, 0)),
                   pl.BlockSpec((TB, m), lambda b: (b, 0)),
                   pl.BlockSpec((8, TB), lambda b: (0, b)),
                   pl.BlockSpec((8, kd), lambda b: (0, 0)),
                   pl.BlockSpec((8, kd), lambda b: (0, 0))],
        out_shape=[jax.ShapeDtypeStruct((N, m), jnp.float32),
                   jax.ShapeDtypeStruct((N, m), jnp.bfloat16),
                   jax.ShapeDtypeStruct((8, N), jnp.float32),
                   jax.ShapeDtypeStruct((8, kd), jnp.float32),
                   jax.ShapeDtypeStruct((8, kd), jnp.float32)],
    )(mt, s, kp, x, w)


def _aggdeg(mt, kp):
    return pl.pallas_call(
        _aggdeg_kernel,
        grid=(N // TB,),
        in_specs=[pl.BlockSpec((TB, N), lambda b: (b, 0)),
                  pl.BlockSpec((8, N), lambda b: (0, 0))],
        out_specs=pl.BlockSpec((8, TB), lambda b: (0, b)),
        out_shape=jax.ShapeDtypeStruct((8, N), jnp.float32),
    )(mt, kp)


def _readout(h, s, kp, k):
    mx, sm = pl.pallas_call(
        _readout_kernel,
        grid=(NT,),
        in_specs=[pl.BlockSpec((T, C), lambda i: (i, 0)),
                  pl.BlockSpec((8, N), lambda i: (0, 0)),
                  pl.BlockSpec((8, N), lambda i: (0, 0))],
        out_specs=[pl.BlockSpec((8, C), lambda i: (0, 0)),
                   pl.BlockSpec((8, C), lambda i: (0, 0))],
        out_shape=[jax.ShapeDtypeStruct((8, C), jnp.float32),
                   jax.ShapeDtypeStruct((8, C), jnp.float32)],
    )(h, s, kp)
    return jnp.concatenate([mx[0], sm[0] / k])


def _pool(sc, kx):
    tau = jax.lax.top_k(sc, kx)[0][kx - 1]
    keep_n = (sc >= tau).astype(jnp.float32)
    s_n = keep_n * jnp.tanh(sc)
    s8 = jnp.broadcast_to(s_n[None, :], (8, N))
    k8 = jnp.broadcast_to(keep_n[None, :], (8, N))
    return s8, k8


def kernel(feature, W1, b1, Wp1, bp1, W2, b2, Wp2, bp2, W3, b3, Wp3, bp3):
    f32 = jnp.float32
    fp = jnp.pad(feature.astype(f32), ((0, 0), (0, KF - feature.shape[1])))
    w1p = jnp.pad(W1, ((0, KF - W1.shape[0]), (0, 0)))

    d, mx = _dist(fp)
    t = 0.5 * mx[0, 0]
    mt, dv1 = _mask(d, t)

    ones8 = jnp.ones((8, N), f32)
    k1, k2, k3 = 3072, 2304, 1728

    xw1, u1, _, _ = _lin(fp, w1p, ones8, dv1, ones8)
    h1, sw1 = _aggconv(mt, u1, xw1, dv1, Wp1, b1)
    sc1 = _aggscore(mt, sw1, dv1, bp1)[0]
    s1, kp2 = _pool(sc1, k1)

    xw2, u2, dv2, mx1, sm1 = _deglin(mt, h1, W2, s1, kp2)
    x1r = jnp.concatenate([mx1[0], sm1[0] / k1])
    h2, sw2 = _aggconv(mt, u2, xw2, dv2, Wp2, b2)
    sc2 = _aggscore(mt, sw2, dv2, bp2)[0]
    s2, kp3 = _pool(sc2, k2)

    xw3, u3, dv3, mx2, sm2 = _deglin(mt, h2, W3, s2, kp3)
    x2r = jnp.concatenate([mx2[0], sm2[0] / k2])
    h3, sw3 = _aggconv(mt, u3, xw3, dv3, Wp3, b3)
    sc3 = _aggscore(mt, sw3, dv3, bp3)[0]
    s3, kp4 = _pool(sc3, k3)
    x3r = _readout(h3, s3, kp4, k3)

    return (x1r + x2r + x3r)[None, :]
